# Initial kernel scaffold; baseline (speedup 1.0000x reference)
import jax
import jax.numpy as jnp
from jax.experimental import pallas as pl


def kernel(inputs, edge_index, batch, conv_weight, gru_w_ih, gru_w_hh, gru_b_ih, gru_b_hh, W1, b1, W2, b2):
    raise NotImplementedError("write your pallas kernel here")


# speedup vs baseline: 4.5141x; 4.5141x over previous
import functools

import jax
import jax.numpy as jnp
from jax import lax
from jax.experimental import pallas as pl
from jax.experimental.pallas import tpu as pltpu
from jax.experimental.pallas import tpu_sc as plsc

N_NODES = 10000
N_EDGES = 320000
D = 128
NUM_LAYERS = 3
NUM_GRAPHS = 64
NUM_CLASSES = 10

N_PAD = 10240
E_PER_TILE = N_EDGES // 32
CHUNK = 80
N_CHUNKS = E_PER_TILE // CHUNK
ROWS_PER_TILE = N_PAD // 16


def _sc_agg_body(m_hbm, src_hbm, dst_hbm, out_hbm,
                 acc, sidx, didx, rowbuf, sem):
    cid = lax.axis_index("c")
    sid = lax.axis_index("s")

    def _zrow(i, carry):
        rowbuf[i // 8, pl.ds((i % 8) * 16, 16)] = jnp.zeros((16,), jnp.float32)
        return carry
    lax.fori_loop(0, CHUNK * 8, _zrow, 0)

    def _zacc(i, carry):
        pltpu.sync_copy(rowbuf, acc.at[pl.ds(sid * ROWS_PER_TILE + i * CHUNK, CHUNK)])
        return carry
    lax.fori_loop(0, ROWS_PER_TILE // CHUNK, _zacc, 0)
    plsc.subcore_barrier()

    wid = sid * 2 + cid
    ebase = wid * E_PER_TILE

    def _step(j, carry):
        off = ebase + j * CHUNK
        pltpu.sync_copy(src_hbm.at[pl.ds(off, CHUNK)], sidx)
        pltpu.sync_copy(dst_hbm.at[pl.ds(off, CHUNK)], didx)
        pltpu.async_copy(m_hbm.at[sidx], rowbuf, sem).wait()
        pltpu.sync_copy(rowbuf, acc.at[didx], add=True)
        return carry
    lax.fori_loop(0, N_CHUNKS, _step, 0)
    plsc.subcore_barrier()

    def _dump(i, carry):
        r = sid * ROWS_PER_TILE + i * CHUNK
        pltpu.sync_copy(acc.at[pl.ds(r, CHUNK)], out_hbm.at[cid, pl.ds(r, CHUNK)])
        return carry
    lax.fori_loop(0, ROWS_PER_TILE // CHUNK, _dump, 0)


@functools.lru_cache(maxsize=None)
def _get_sc_agg():
    return pl.kernel(
        _sc_agg_body,
        out_type=jax.ShapeDtypeStruct((2, N_PAD, D), jnp.float32),
        mesh=plsc.VectorSubcoreMesh(core_axis_name="c", subcore_axis_name="s"),
        scratch_types=[
            pltpu.VMEM_SHARED((N_PAD, D), jnp.float32),
            pltpu.VMEM((CHUNK,), jnp.int32),
            pltpu.VMEM((CHUNK,), jnp.int32),
            pltpu.VMEM((CHUNK, D), jnp.float32),
            pltpu.SemaphoreType.DMA,
        ],
    )


_ROWS = 1000


def _mm_body(x_ref, w_ref, o_ref):
    o_ref[...] = jnp.dot(x_ref[...], w_ref[...], preferred_element_type=jnp.float32)


def _first_msg(x, w):
    return pl.pallas_call(
        _mm_body,
        grid=(N_NODES // _ROWS,),
        in_specs=[pl.BlockSpec((_ROWS, D), lambda i: (i, 0)),
                  pl.BlockSpec((D, D), lambda i: (0, 0))],
        out_specs=pl.BlockSpec((_ROWS, D), lambda i: (i, 0)),
        out_shape=jax.ShapeDtypeStruct((N_NODES, D), jnp.float32),
    )(x, w)


def _gru_body(p_ref, x_ref, wih_ref, whh_ref, bih_ref, bhh_ref, wnext_ref,
              xo_ref, *maybe_mo, with_next):
    agg = p_ref[0] + p_ref[1]
    h = x_ref[...]
    gi = jnp.dot(agg, wih_ref[...], preferred_element_type=jnp.float32) + bih_ref[...]
    gh = jnp.dot(h, whh_ref[...], preferred_element_type=jnp.float32) + bhh_ref[...]
    r = jax.nn.sigmoid(gi[:, :D] + gh[:, :D])
    z = jax.nn.sigmoid(gi[:, D:2 * D] + gh[:, D:2 * D])
    n = jnp.tanh(gi[:, 2 * D:] + r * gh[:, 2 * D:])
    xn = (1.0 - z) * n + z * h
    xo_ref[...] = xn
    if with_next:
        maybe_mo[0][...] = jnp.dot(xn, wnext_ref[...], preferred_element_type=jnp.float32)


def _gru_layer(parts, x, wih_t, whh_t, bih, bhh, w_next, with_next):
    out_shapes = [jax.ShapeDtypeStruct((N_NODES, D), jnp.float32)]
    out_specs = [pl.BlockSpec((_ROWS, D), lambda i: (i, 0))]
    if with_next:
        out_shapes.append(jax.ShapeDtypeStruct((N_NODES, D), jnp.float32))
        out_specs.append(pl.BlockSpec((_ROWS, D), lambda i: (i, 0)))
    res = pl.pallas_call(
        functools.partial(_gru_body, with_next=with_next),
        grid=(N_NODES // _ROWS,),
        in_specs=[
            pl.BlockSpec((2, _ROWS, D), lambda i: (0, i, 0)),
            pl.BlockSpec((_ROWS, D), lambda i: (i, 0)),
            pl.BlockSpec((D, 3 * D), lambda i: (0, 0)),
            pl.BlockSpec((D, 3 * D), lambda i: (0, 0)),
            pl.BlockSpec((1, 3 * D), lambda i: (0, 0)),
            pl.BlockSpec((1, 3 * D), lambda i: (0, 0)),
            pl.BlockSpec((D, D), lambda i: (0, 0)),
        ],
        out_specs=out_specs,
        out_shape=out_shapes,
    )(parts, x, wih_t, whh_t, bih, bhh, w_next)
    return (res[0], res[1]) if with_next else (res[0], None)


def _pool_mlp_body(x_ref, b_ref, w1t_ref, b1_ref, w2t_ref, b2_ref, o_ref, pooled):
    bvec = b_ref[...]
    x = x_ref[...]

    def body(g, carry):
        vals = jnp.where(bvec == g, x, -jnp.inf)
        pooled[pl.ds(g, 1), :] = jnp.max(vals, axis=0, keepdims=True)
        return carry
    lax.fori_loop(0, NUM_GRAPHS, body, 0)
    p = pooled[...]
    h = jnp.maximum(jnp.dot(p, w1t_ref[...], preferred_element_type=jnp.float32)
                    + b1_ref[...], 0.0)
    o_ref[...] = jnp.dot(h, w2t_ref[...], preferred_element_type=jnp.float32) + b2_ref[...]


def _pool_mlp(x, batch2d, w1t, b1, w2t_pad, b2_pad):
    return pl.pallas_call(
        _pool_mlp_body,
        out_shape=jax.ShapeDtypeStruct((NUM_GRAPHS, D), jnp.float32),
        scratch_shapes=[pltpu.VMEM((NUM_GRAPHS, D), jnp.float32)],
    )(x, batch2d, w1t, b1, w2t_pad, b2_pad)


def kernel(inputs, edge_index, batch, conv_weight, gru_w_ih, gru_w_hh,
           gru_b_ih, gru_b_hh, W1, b1, W2, b2):
    src = edge_index[0]
    dst = edge_index[1]
    wih_t = gru_w_ih.T
    whh_t = gru_w_hh.T
    bih = gru_b_ih.reshape(1, 3 * D)
    bhh = gru_b_hh.reshape(1, 3 * D)
    w1t = W1.T
    b1r = b1.reshape(1, 4 * D)
    w2t_pad = jnp.zeros((4 * D, D), jnp.float32).at[:, :NUM_CLASSES].set(W2.T)
    b2_pad = jnp.zeros((1, D), jnp.float32).at[0, :NUM_CLASSES].set(b2)

    x = inputs
    m = _first_msg(x, conv_weight[0])
    for i in range(NUM_LAYERS):
        parts = _get_sc_agg()(m, src, dst)[:, :N_NODES, :]
        with_next = i + 1 < NUM_LAYERS
        w_next = conv_weight[i + 1] if with_next else conv_weight[0]
        x, m = _gru_layer(parts, x, wih_t, whh_t, bih, bhh, w_next, with_next)

    batch2d = batch.reshape(N_NODES, 1)
    out_pad = _pool_mlp(x, batch2d, w1t, b1r, w2t_pad, b2_pad)
    return out_pad[:, :NUM_CLASSES]


# speedup vs baseline: 10.3054x; 2.2830x over previous
import functools

import jax
import jax.numpy as jnp
from jax import lax
from jax.experimental import pallas as pl
from jax.experimental.pallas import tpu as pltpu
from jax.experimental.pallas import tpu_sc as plsc

N_NODES = 10000
N_EDGES = 320000
D = 128
NUM_LAYERS = 3
NUM_GRAPHS = 64
NUM_CLASSES = 10

N_PAD = 10240
E_PER_TILE = N_EDGES // 32
CHUNK = 80
N_CHUNKS = E_PER_TILE // CHUNK
ROWS_PER_TILE = N_PAD // 16


NBUF = 4
IBUF = 8


def _sc_agg_body(m_hbm, e3_hbm, out_hbm, acc, eidx, rowbuf, *sems):
    isem = sems[:IBUF]
    gsem = sems[IBUF:]
    cid = lax.axis_index("c")
    sid = lax.axis_index("s")
    wid = sid * 2 + cid

    def _zrow(i, carry):
        rowbuf[0, i // 8, pl.ds((i % 8) * 16, 16)] = jnp.zeros((16,), jnp.float32)
        return carry
    lax.fori_loop(0, CHUNK * 8, _zrow, 0)

    def _zacc(i, carry):
        pltpu.sync_copy(rowbuf.at[0],
                        acc.at[pl.ds(sid * ROWS_PER_TILE + i * CHUNK, CHUNK)])
        return carry
    lax.fori_loop(0, ROWS_PER_TILE // CHUNK, _zacc, 0)
    plsc.subcore_barrier()

    def _idx_load(c, bi):
        pltpu.async_copy(e3_hbm.at[wid, c], eidx.at[bi], isem[bi])

    def _idx_wait(bi):
        pltpu.make_async_copy(e3_hbm.at[wid, 0], eidx.at[bi], isem[bi]).wait()

    def _gather(c, bi, b):
        pltpu.async_copy(m_hbm.at[eidx.at[bi, 0]], rowbuf.at[b], gsem[b])

    def _gather_wait(b):
        pltpu.make_async_copy(m_hbm.at[eidx.at[0, 0]], rowbuf.at[b],
                              gsem[b]).wait()

    for i in range(IBUF):
        _idx_load(i, i)
    for b in range(NBUF):
        _idx_wait(b)
        _gather(b, b, b)

    def _body(c, k):
        b = k % NBUF
        bi = k % IBUF
        _gather_wait(b)
        pltpu.sync_copy(rowbuf.at[b], acc.at[eidx.at[bi, 1]], add=True)
        return b, bi

    def _super(t, carry):
        for k in range(IBUF):
            c = t * IBUF + k
            b, bi = _body(c, k)
            p = c + IBUF

            @pl.when(p < N_CHUNKS)
            def _pf_idx():
                _idx_load(p, bi)

            g = c + NBUF

            @pl.when(g < N_CHUNKS)
            def _pf_gather():
                _idx_wait((k + NBUF) % IBUF)
                _gather(g, (k + NBUF) % IBUF, b)
        return carry
    lax.fori_loop(0, (N_CHUNKS // IBUF), _super, 0)

    for c in range(IBUF * (N_CHUNKS // IBUF), N_CHUNKS):
        k = c % IBUF
        b, bi = _body(c, k)
        g = c + NBUF
        if g < N_CHUNKS:
            _idx_wait((k + NBUF) % IBUF)
            _gather(g, (k + NBUF) % IBUF, b)
    plsc.subcore_barrier()

    def _dump(i, carry):
        r = sid * ROWS_PER_TILE + i * CHUNK
        pltpu.sync_copy(acc.at[pl.ds(r, CHUNK)], out_hbm.at[cid, pl.ds(r, CHUNK)])
        return carry
    lax.fori_loop(0, ROWS_PER_TILE // CHUNK, _dump, 0)


@functools.lru_cache(maxsize=None)
def _get_sc_agg():
    return pl.kernel(
        _sc_agg_body,
        out_type=jax.ShapeDtypeStruct((2, N_PAD, D), jnp.float32),
        mesh=plsc.VectorSubcoreMesh(core_axis_name="c", subcore_axis_name="s"),
        scratch_types=[
            pltpu.VMEM_SHARED((N_PAD, D), jnp.float32),
            pltpu.VMEM((IBUF, 2, CHUNK), jnp.int32),
            pltpu.VMEM((NBUF, CHUNK, D), jnp.float32),
        ] + [pltpu.SemaphoreType.DMA] * (IBUF + NBUF),
    )


_ROWS = 1000


def _mm_body(x_ref, w_ref, o_ref):
    o_ref[...] = jnp.dot(x_ref[...], w_ref[...], preferred_element_type=jnp.float32)


def _first_msg(x, w):
    return pl.pallas_call(
        _mm_body,
        grid=(N_NODES // _ROWS,),
        in_specs=[pl.BlockSpec((_ROWS, D), lambda i: (i, 0)),
                  pl.BlockSpec((D, D), lambda i: (0, 0))],
        out_specs=pl.BlockSpec((_ROWS, D), lambda i: (i, 0)),
        out_shape=jax.ShapeDtypeStruct((N_NODES, D), jnp.float32),
    )(x, w)


def _gru_body(p_ref, x_ref, wih_ref, whh_ref, bih_ref, bhh_ref, wnext_ref,
              xo_ref, *maybe_mo, with_next):
    agg = p_ref[0] + p_ref[1]
    h = x_ref[...]
    gi = jnp.dot(agg, wih_ref[...], preferred_element_type=jnp.float32) + bih_ref[...]
    gh = jnp.dot(h, whh_ref[...], preferred_element_type=jnp.float32) + bhh_ref[...]
    r = jax.nn.sigmoid(gi[:, :D] + gh[:, :D])
    z = jax.nn.sigmoid(gi[:, D:2 * D] + gh[:, D:2 * D])
    n = jnp.tanh(gi[:, 2 * D:] + r * gh[:, 2 * D:])
    xn = (1.0 - z) * n + z * h
    xo_ref[...] = xn
    if with_next:
        maybe_mo[0][...] = jnp.dot(xn, wnext_ref[...], preferred_element_type=jnp.float32)


def _gru_layer(parts, x, wih_t, whh_t, bih, bhh, w_next, with_next):
    out_shapes = [jax.ShapeDtypeStruct((N_NODES, D), jnp.float32)]
    out_specs = [pl.BlockSpec((_ROWS, D), lambda i: (i, 0))]
    if with_next:
        out_shapes.append(jax.ShapeDtypeStruct((N_NODES, D), jnp.float32))
        out_specs.append(pl.BlockSpec((_ROWS, D), lambda i: (i, 0)))
    res = pl.pallas_call(
        functools.partial(_gru_body, with_next=with_next),
        grid=(N_NODES // _ROWS,),
        in_specs=[
            pl.BlockSpec((2, _ROWS, D), lambda i: (0, i, 0)),
            pl.BlockSpec((_ROWS, D), lambda i: (i, 0)),
            pl.BlockSpec((D, 3 * D), lambda i: (0, 0)),
            pl.BlockSpec((D, 3 * D), lambda i: (0, 0)),
            pl.BlockSpec((1, 3 * D), lambda i: (0, 0)),
            pl.BlockSpec((1, 3 * D), lambda i: (0, 0)),
            pl.BlockSpec((D, D), lambda i: (0, 0)),
        ],
        out_specs=out_specs,
        out_shape=out_shapes,
    )(parts, x, wih_t, whh_t, bih, bhh, w_next)
    return (res[0], res[1]) if with_next else (res[0], None)


def _pool_mlp_body(x_ref, b_ref, w1t_ref, b1_ref, w2t_ref, b2_ref, o_ref, pooled):
    bvec = b_ref[...]
    x = x_ref[...]

    def body(g, carry):
        vals = jnp.where(bvec == g, x, -jnp.inf)
        pooled[pl.ds(g, 1), :] = jnp.max(vals, axis=0, keepdims=True)
        return carry
    lax.fori_loop(0, NUM_GRAPHS, body, 0)
    p = pooled[...]
    h = jnp.maximum(jnp.dot(p, w1t_ref[...], preferred_element_type=jnp.float32)
                    + b1_ref[...], 0.0)
    o_ref[...] = jnp.dot(h, w2t_ref[...], preferred_element_type=jnp.float32) + b2_ref[...]


def _pool_mlp(x, batch2d, w1t, b1, w2t_pad, b2_pad):
    return pl.pallas_call(
        _pool_mlp_body,
        out_shape=jax.ShapeDtypeStruct((NUM_GRAPHS, D), jnp.float32),
        scratch_shapes=[pltpu.VMEM((NUM_GRAPHS, D), jnp.float32)],
    )(x, batch2d, w1t, b1, w2t_pad, b2_pad)


def kernel(inputs, edge_index, batch, conv_weight, gru_w_ih, gru_w_hh,
           gru_b_ih, gru_b_hh, W1, b1, W2, b2):
    e3 = jnp.stack([edge_index[0].reshape(32, N_CHUNKS, CHUNK),
                    edge_index[1].reshape(32, N_CHUNKS, CHUNK)], axis=2)
    wih_t = gru_w_ih.T
    whh_t = gru_w_hh.T
    bih = gru_b_ih.reshape(1, 3 * D)
    bhh = gru_b_hh.reshape(1, 3 * D)
    w1t = W1.T
    b1r = b1.reshape(1, 4 * D)
    w2t_pad = jnp.zeros((4 * D, D), jnp.float32).at[:, :NUM_CLASSES].set(W2.T)
    b2_pad = jnp.zeros((1, D), jnp.float32).at[0, :NUM_CLASSES].set(b2)

    x = inputs
    m = _first_msg(x, conv_weight[0])
    for i in range(NUM_LAYERS):
        parts = _get_sc_agg()(m, e3)[:, :N_NODES, :]
        with_next = i + 1 < NUM_LAYERS
        w_next = conv_weight[i + 1] if with_next else conv_weight[0]
        x, m = _gru_layer(parts, x, wih_t, whh_t, bih, bhh, w_next, with_next)

    batch2d = batch.reshape(N_NODES, 1)
    out_pad = _pool_mlp(x, batch2d, w1t, b1r, w2t_pad, b2_pad)
    return out_pad[:, :NUM_CLASSES]


# speedup vs baseline: 10.7726x; 1.0453x over previous
import functools

import jax
import jax.numpy as jnp
from jax import lax
from jax.experimental import pallas as pl
from jax.experimental.pallas import tpu as pltpu
from jax.experimental.pallas import tpu_sc as plsc

N_NODES = 10000
N_EDGES = 320000
D = 128
NUM_LAYERS = 3
NUM_GRAPHS = 64
NUM_CLASSES = 10

N_PAD = 10240
E_PER_TILE = N_EDGES // 32
CHUNK = 80
N_CHUNKS = E_PER_TILE // CHUNK
ROWS_PER_TILE = N_PAD // 16


NBUF = 4
IBUF = 8


def _sc_agg_body(m_hbm, e3_hbm, out_hbm, acc, eidx, rowbuf, *sems):
    isem = sems[:IBUF]
    gsem = sems[IBUF:]
    cid = lax.axis_index("c")
    sid = lax.axis_index("s")
    wid = sid * 2 + cid

    def _zrow(i, carry):
        rowbuf[0, i // 8, pl.ds((i % 8) * 16, 16)] = jnp.zeros((16,), jnp.float32)
        return carry
    lax.fori_loop(0, CHUNK * 8, _zrow, 0)

    def _zacc(i, carry):
        pltpu.sync_copy(rowbuf.at[0],
                        acc.at[pl.ds(sid * ROWS_PER_TILE + i * CHUNK, CHUNK)])
        return carry
    lax.fori_loop(0, ROWS_PER_TILE // CHUNK, _zacc, 0)
    plsc.subcore_barrier()

    def _idx_load(c, bi):
        pltpu.async_copy(e3_hbm.at[wid, c], eidx.at[bi], isem[bi])

    def _idx_wait(bi):
        pltpu.make_async_copy(e3_hbm.at[wid, 0], eidx.at[bi], isem[bi]).wait()

    def _gather(c, bi, b):
        pltpu.async_copy(m_hbm.at[eidx.at[bi, 0]], rowbuf.at[b], gsem[b])

    def _gather_wait(b):
        pltpu.make_async_copy(m_hbm.at[eidx.at[0, 0]], rowbuf.at[b],
                              gsem[b]).wait()

    for i in range(IBUF):
        _idx_load(i, i)
    for b in range(NBUF):
        _idx_wait(b)
        _gather(b, b, b)

    def _body(c, k):
        b = k % NBUF
        bi = k % IBUF
        _gather_wait(b)
        pltpu.sync_copy(rowbuf.at[b], acc.at[eidx.at[bi, 1]], add=True)
        return b, bi

    def _super(t, carry):
        for k in range(IBUF):
            c = t * IBUF + k
            b, bi = _body(c, k)
            p = c + IBUF

            @pl.when(p < N_CHUNKS)
            def _pf_idx():
                _idx_load(p, bi)

            g = c + NBUF

            @pl.when(g < N_CHUNKS)
            def _pf_gather():
                _idx_wait((k + NBUF) % IBUF)
                _gather(g, (k + NBUF) % IBUF, b)
        return carry
    lax.fori_loop(0, (N_CHUNKS // IBUF), _super, 0)

    for c in range(IBUF * (N_CHUNKS // IBUF), N_CHUNKS):
        k = c % IBUF
        b, bi = _body(c, k)
        g = c + NBUF
        if g < N_CHUNKS:
            _idx_wait((k + NBUF) % IBUF)
            _gather(g, (k + NBUF) % IBUF, b)
    plsc.subcore_barrier()

    def _dump(i, carry):
        r = sid * ROWS_PER_TILE + i * CHUNK
        pltpu.sync_copy(acc.at[pl.ds(r, CHUNK)], out_hbm.at[cid, pl.ds(r, CHUNK)])
        return carry
    lax.fori_loop(0, ROWS_PER_TILE // CHUNK, _dump, 0)


@functools.lru_cache(maxsize=None)
def _get_sc_agg():
    return pl.kernel(
        _sc_agg_body,
        out_type=jax.ShapeDtypeStruct((2, N_PAD, D), jnp.float32),
        mesh=plsc.VectorSubcoreMesh(core_axis_name="c", subcore_axis_name="s"),
        scratch_types=[
            pltpu.VMEM_SHARED((N_PAD, D), jnp.float32),
            pltpu.VMEM((IBUF, 2, CHUNK), jnp.int32),
            pltpu.VMEM((NBUF, CHUNK, D), jnp.float32),
        ] + [pltpu.SemaphoreType.DMA] * (IBUF + NBUF),
    )


_ROWS = 1000


def _mm_body(x_ref, w_ref, o_ref):
    o_ref[...] = jnp.dot(x_ref[...], w_ref[...], preferred_element_type=jnp.float32)


def _first_msg(x, w):
    return pl.pallas_call(
        _mm_body,
        grid=(N_NODES // _ROWS,),
        in_specs=[pl.BlockSpec((_ROWS, D), lambda i: (i, 0)),
                  pl.BlockSpec((D, D), lambda i: (0, 0))],
        out_specs=pl.BlockSpec((_ROWS, D), lambda i: (i, 0)),
        out_shape=jax.ShapeDtypeStruct((N_NODES, D), jnp.float32),
    )(x, w)


def _gru_body(p_ref, x_ref, wih_ref, whh_ref, bih_ref, bhh_ref, wnext_ref,
              xo_ref, *maybe_mo, with_next):
    agg = p_ref[0] + p_ref[1]
    h = x_ref[...]
    gi = jnp.dot(agg, wih_ref[...], preferred_element_type=jnp.float32) + bih_ref[...]
    gh = jnp.dot(h, whh_ref[...], preferred_element_type=jnp.float32) + bhh_ref[...]
    r = jax.nn.sigmoid(gi[:, :D] + gh[:, :D])
    z = jax.nn.sigmoid(gi[:, D:2 * D] + gh[:, D:2 * D])
    n = jnp.tanh(gi[:, 2 * D:] + r * gh[:, 2 * D:])
    xn = (1.0 - z) * n + z * h
    xo_ref[...] = xn
    if with_next:
        maybe_mo[0][...] = jnp.dot(xn, wnext_ref[...], preferred_element_type=jnp.float32)


def _gru_layer(parts, x, wih_t, whh_t, bih, bhh, w_next, with_next):
    out_shapes = [jax.ShapeDtypeStruct((N_NODES, D), jnp.float32)]
    out_specs = [pl.BlockSpec((_ROWS, D), lambda i: (i, 0))]
    if with_next:
        out_shapes.append(jax.ShapeDtypeStruct((N_NODES, D), jnp.float32))
        out_specs.append(pl.BlockSpec((_ROWS, D), lambda i: (i, 0)))
    res = pl.pallas_call(
        functools.partial(_gru_body, with_next=with_next),
        grid=(N_NODES // _ROWS,),
        in_specs=[
            pl.BlockSpec((2, _ROWS, D), lambda i: (0, i, 0)),
            pl.BlockSpec((_ROWS, D), lambda i: (i, 0)),
            pl.BlockSpec((D, 3 * D), lambda i: (0, 0)),
            pl.BlockSpec((D, 3 * D), lambda i: (0, 0)),
            pl.BlockSpec((1, 3 * D), lambda i: (0, 0)),
            pl.BlockSpec((1, 3 * D), lambda i: (0, 0)),
            pl.BlockSpec((D, D), lambda i: (0, 0)),
        ],
        out_specs=out_specs,
        out_shape=out_shapes,
    )(parts, x, wih_t, whh_t, bih, bhh, w_next)
    return (res[0], res[1]) if with_next else (res[0], None)


def _pool_mlp_body(x_ref, b_ref, w1t_ref, b1_ref, w2t_ref, b2_ref, o_ref, pooled):
    bvec = b_ref[...]
    x = x_ref[...]

    def body(g, carry):
        vals = jnp.where(bvec == g, x, -jnp.inf)
        pooled[pl.ds(g, 1), :] = jnp.max(vals, axis=0, keepdims=True)
        return carry
    lax.fori_loop(0, NUM_GRAPHS, body, 0)
    p = pooled[...]
    h = jnp.maximum(jnp.dot(p, w1t_ref[...], preferred_element_type=jnp.float32)
                    + b1_ref[...], 0.0)
    o_ref[...] = jnp.dot(h, w2t_ref[...], preferred_element_type=jnp.float32) + b2_ref[...]


def _pool_mlp(x, batch2d, w1t, b1, w2t_pad, b2_pad):
    return pl.pallas_call(
        _pool_mlp_body,
        out_shape=jax.ShapeDtypeStruct((NUM_GRAPHS, D), jnp.float32),
        scratch_shapes=[pltpu.VMEM((NUM_GRAPHS, D), jnp.float32)],
    )(x, batch2d, w1t, b1, w2t_pad, b2_pad)


def kernel(inputs, edge_index, batch, conv_weight, gru_w_ih, gru_w_hh,
           gru_b_ih, gru_b_hh, W1, b1, W2, b2):
    e3 = jnp.stack([edge_index[0].reshape(32, N_CHUNKS, CHUNK),
                    edge_index[1].reshape(32, N_CHUNKS, CHUNK)], axis=2)
    wih_t = gru_w_ih.T
    whh_t = gru_w_hh.T
    bih = gru_b_ih.reshape(1, 3 * D)
    bhh = gru_b_hh.reshape(1, 3 * D)
    w1t = W1.T
    b1r = b1.reshape(1, 4 * D)
    w2t_pad = jnp.zeros((4 * D, D), jnp.float32).at[:, :NUM_CLASSES].set(W2.T)
    b2_pad = jnp.zeros((1, D), jnp.float32).at[0, :NUM_CLASSES].set(b2)

    x = inputs
    m = _first_msg(x, conv_weight[0])
    for i in range(NUM_LAYERS):
        parts = _get_sc_agg()(m, e3)
        with_next = i + 1 < NUM_LAYERS
        w_next = conv_weight[i + 1] if with_next else conv_weight[0]
        x, m = _gru_layer(parts, x, wih_t, whh_t, bih, bhh, w_next, with_next)

    batch2d = batch.reshape(N_NODES, 1)
    out_pad = _pool_mlp(x, batch2d, w1t, b1r, w2t_pad, b2_pad)
    return out_pad[:, :NUM_CLASSES]


# speedup vs baseline: 12.5944x; 1.1691x over previous
import functools

import jax
import jax.numpy as jnp
from jax import lax
from jax.experimental import pallas as pl
from jax.experimental.pallas import tpu as pltpu
from jax.experimental.pallas import tpu_sc as plsc

N_NODES = 10000
N_EDGES = 320000
D = 128
NUM_LAYERS = 3
NUM_GRAPHS = 64
NUM_CLASSES = 10

N_PAD = 10240
E_PER_TILE = N_EDGES // 32
CHUNK = 80
N_CHUNKS = E_PER_TILE // CHUNK
ROWS_PER_TILE = N_PAD // 16


NBUF = 4
IBUF = 8


def _sc_agg_body(m_hbm, e3_hbm, out_hbm, acc, eidx, rowbuf, *sems):
    isem = sems[:IBUF]
    gsem = sems[IBUF:]
    cid = lax.axis_index("c")
    sid = lax.axis_index("s")
    wid = sid * 2 + cid

    def _zrow(i, carry):
        rowbuf[0, i // 8, pl.ds((i % 8) * 16, 16)] = jnp.zeros((16,), jnp.float32)
        return carry
    lax.fori_loop(0, CHUNK * 8, _zrow, 0)

    def _zacc(i, carry):
        pltpu.sync_copy(rowbuf.at[0],
                        acc.at[pl.ds(sid * ROWS_PER_TILE + i * CHUNK, CHUNK)])
        return carry
    lax.fori_loop(0, ROWS_PER_TILE // CHUNK, _zacc, 0)
    plsc.subcore_barrier()

    def _idx_load(c, bi):
        pltpu.async_copy(e3_hbm.at[wid, c], eidx.at[bi], isem[bi])

    def _idx_wait(bi):
        pltpu.make_async_copy(e3_hbm.at[wid, 0], eidx.at[bi], isem[bi]).wait()

    def _gather(c, bi, b):
        pltpu.async_copy(m_hbm.at[eidx.at[bi, 0]], rowbuf.at[b], gsem[b])

    def _gather_wait(b):
        pltpu.make_async_copy(m_hbm.at[eidx.at[0, 0]], rowbuf.at[b],
                              gsem[b]).wait()

    for i in range(IBUF):
        _idx_load(i, i)
    for b in range(NBUF):
        _idx_wait(b)
        _gather(b, b, b)

    def _body(c, k):
        b = k % NBUF
        bi = k % IBUF
        _gather_wait(b)
        pltpu.sync_copy(rowbuf.at[b], acc.at[eidx.at[bi, 1]], add=True)
        return b, bi

    def _super(t, carry):
        for k in range(IBUF):
            c = t * IBUF + k
            b, bi = _body(c, k)
            p = c + IBUF

            @pl.when(p < N_CHUNKS)
            def _pf_idx():
                _idx_load(p, bi)

            g = c + NBUF

            @pl.when(g < N_CHUNKS)
            def _pf_gather():
                _idx_wait((k + NBUF) % IBUF)
                _gather(g, (k + NBUF) % IBUF, b)
        return carry
    lax.fori_loop(0, (N_CHUNKS // IBUF), _super, 0)

    for c in range(IBUF * (N_CHUNKS // IBUF), N_CHUNKS):
        k = c % IBUF
        b, bi = _body(c, k)
        g = c + NBUF
        if g < N_CHUNKS:
            _idx_wait((k + NBUF) % IBUF)
            _gather(g, (k + NBUF) % IBUF, b)
    plsc.subcore_barrier()

    def _dump(i, carry):
        r = sid * ROWS_PER_TILE + i * CHUNK
        pltpu.sync_copy(acc.at[pl.ds(r, CHUNK)], out_hbm.at[cid, pl.ds(r, CHUNK)])
        return carry
    lax.fori_loop(0, ROWS_PER_TILE // CHUNK, _dump, 0)


@functools.lru_cache(maxsize=None)
def _get_sc_agg():
    return pl.kernel(
        _sc_agg_body,
        out_type=jax.ShapeDtypeStruct((2, N_PAD, D), jnp.float32),
        mesh=plsc.VectorSubcoreMesh(core_axis_name="c", subcore_axis_name="s"),
        scratch_types=[
            pltpu.VMEM_SHARED((N_PAD, D), jnp.float32),
            pltpu.VMEM((IBUF, 2, CHUNK), jnp.int32),
            pltpu.VMEM((NBUF, CHUNK, D), jnp.float32),
        ] + [pltpu.SemaphoreType.DMA] * (IBUF + NBUF),
    )


_ROWS = 1000


def _mm_body(x_ref, w_ref, o_ref):
    o_ref[...] = jnp.dot(x_ref[...], w_ref[...], preferred_element_type=jnp.float32)


def _first_msg(x, w):
    return pl.pallas_call(
        _mm_body,
        grid=(N_NODES // _ROWS,),
        in_specs=[pl.BlockSpec((_ROWS, D), lambda i: (i, 0)),
                  pl.BlockSpec((D, D), lambda i: (0, 0))],
        out_specs=pl.BlockSpec((_ROWS, D), lambda i: (i, 0)),
        out_shape=jax.ShapeDtypeStruct((N_NODES, D), jnp.float32),
    )(x, w)


def _gru_body(p_ref, x_ref, wih_ref, whh_ref, bih_ref, bhh_ref, wnext_ref,
              xo_ref, *maybe_mo, with_next):
    agg = p_ref[0] + p_ref[1]
    h = x_ref[...]
    gi = jnp.dot(agg, wih_ref[...], preferred_element_type=jnp.float32) + bih_ref[...]
    gh = jnp.dot(h, whh_ref[...], preferred_element_type=jnp.float32) + bhh_ref[...]
    r = jax.nn.sigmoid(gi[:, :D] + gh[:, :D])
    z = jax.nn.sigmoid(gi[:, D:2 * D] + gh[:, D:2 * D])
    n = jnp.tanh(gi[:, 2 * D:] + r * gh[:, 2 * D:])
    xn = (1.0 - z) * n + z * h
    xo_ref[...] = xn
    if with_next:
        maybe_mo[0][...] = jnp.dot(xn, wnext_ref[...], preferred_element_type=jnp.float32)


def _gru_layer(parts, x, wih_t, whh_t, bih, bhh, w_next, with_next):
    out_shapes = [jax.ShapeDtypeStruct((N_NODES, D), jnp.float32)]
    out_specs = [pl.BlockSpec((_ROWS, D), lambda i: (i, 0))]
    if with_next:
        out_shapes.append(jax.ShapeDtypeStruct((N_NODES, D), jnp.float32))
        out_specs.append(pl.BlockSpec((_ROWS, D), lambda i: (i, 0)))
    res = pl.pallas_call(
        functools.partial(_gru_body, with_next=with_next),
        grid=(N_NODES // _ROWS,),
        in_specs=[
            pl.BlockSpec((2, _ROWS, D), lambda i: (0, i, 0)),
            pl.BlockSpec((_ROWS, D), lambda i: (i, 0)),
            pl.BlockSpec((D, 3 * D), lambda i: (0, 0)),
            pl.BlockSpec((D, 3 * D), lambda i: (0, 0)),
            pl.BlockSpec((1, 3 * D), lambda i: (0, 0)),
            pl.BlockSpec((1, 3 * D), lambda i: (0, 0)),
            pl.BlockSpec((D, D), lambda i: (0, 0)),
        ],
        out_specs=out_specs,
        out_shape=out_shapes,
    )(parts, x, wih_t, whh_t, bih, bhh, w_next)
    return (res[0], res[1]) if with_next else (res[0], None)


def _pool_mlp_body(x_ref, b_ref, w1t_ref, b1_ref, w2t_ref, b2_ref, o_ref, pooled):
    i = pl.program_id(0)
    nblk = pl.num_programs(0)

    @pl.when(i == 0)
    def _init():
        pooled[...] = jnp.full((NUM_GRAPHS, D), -jnp.inf, jnp.float32)

    bvec = b_ref[...]
    x = x_ref[...]
    lo = b_ref[0, 0]
    hi = b_ref[_ROWS - 1, 0]

    def body(g, carry):
        @pl.when(jnp.logical_and(g >= lo, g <= hi))
        def _upd():
            vals = jnp.where(bvec == g, x, -jnp.inf)
            cur = pooled[pl.ds(g, 1), :]
            pooled[pl.ds(g, 1), :] = jnp.maximum(
                cur, jnp.max(vals, axis=0, keepdims=True))
        return carry
    lax.fori_loop(0, NUM_GRAPHS, body, 0)

    @pl.when(i == nblk - 1)
    def _mlp():
        p = pooled[...]
        h = jnp.maximum(jnp.dot(p, w1t_ref[...], preferred_element_type=jnp.float32)
                        + b1_ref[...], 0.0)
        o_ref[...] = (jnp.dot(h, w2t_ref[...], preferred_element_type=jnp.float32)
                      + b2_ref[...])


def _pool_mlp(x, batch2d, w1t, b1, w2t_pad, b2_pad):
    return pl.pallas_call(
        _pool_mlp_body,
        grid=(N_NODES // _ROWS,),
        in_specs=[
            pl.BlockSpec((_ROWS, D), lambda i: (i, 0)),
            pl.BlockSpec((_ROWS, 1), lambda i: (i, 0)),
            pl.BlockSpec((D, 4 * D), lambda i: (0, 0)),
            pl.BlockSpec((1, 4 * D), lambda i: (0, 0)),
            pl.BlockSpec((4 * D, D), lambda i: (0, 0)),
            pl.BlockSpec((1, D), lambda i: (0, 0)),
        ],
        out_specs=pl.BlockSpec((NUM_GRAPHS, D), lambda i: (0, 0)),
        out_shape=jax.ShapeDtypeStruct((NUM_GRAPHS, D), jnp.float32),
        scratch_shapes=[pltpu.VMEM((NUM_GRAPHS, D), jnp.float32)],
    )(x, batch2d, w1t, b1, w2t_pad, b2_pad)


def kernel(inputs, edge_index, batch, conv_weight, gru_w_ih, gru_w_hh,
           gru_b_ih, gru_b_hh, W1, b1, W2, b2):
    e3 = jnp.stack([edge_index[0].reshape(32, N_CHUNKS, CHUNK),
                    edge_index[1].reshape(32, N_CHUNKS, CHUNK)], axis=2)
    wih_t = gru_w_ih.T
    whh_t = gru_w_hh.T
    bih = gru_b_ih.reshape(1, 3 * D)
    bhh = gru_b_hh.reshape(1, 3 * D)
    w1t = W1.T
    b1r = b1.reshape(1, 4 * D)
    w2t_pad = jnp.zeros((4 * D, D), jnp.float32).at[:, :NUM_CLASSES].set(W2.T)
    b2_pad = jnp.zeros((1, D), jnp.float32).at[0, :NUM_CLASSES].set(b2)

    x = inputs
    m = _first_msg(x, conv_weight[0])
    for i in range(NUM_LAYERS):
        parts = _get_sc_agg()(m, e3)
        with_next = i + 1 < NUM_LAYERS
        w_next = conv_weight[i + 1] if with_next else conv_weight[0]
        x, m = _gru_layer(parts, x, wih_t, whh_t, bih, bhh, w_next, with_next)

    batch2d = batch.reshape(N_NODES, 1)
    out_pad = _pool_mlp(x, batch2d, w1t, b1r, w2t_pad, b2_pad)
    return out_pad[:, :NUM_CLASSES]
